# baseline (speedup 1.0000x reference)
import functools

import jax
import jax.numpy as jnp
from jax.experimental import pallas as pl
from jax.experimental.pallas import tpu as pltpu
from jax.experimental.pallas import tpu_sc as plsc


def _logits_kernel(x_ref, wr_ref, br_ref, s_ref):
    s = jax.lax.dot_general(wr_ref[...], x_ref[...], (((0,), (1,)), ((), ())),
                            preferred_element_type=jnp.float32)
    s_ref[...] = s + br_ref[...]


def _logits_t(x, Wr, br):
    BT, D = x.shape
    E = Wr.shape[1]
    return pl.pallas_call(
        _logits_kernel,
        out_shape=jax.ShapeDtypeStruct((E, BT), jnp.float32),
    )(x, Wr, br.reshape(E, 1))


_SC_INFO = plsc.get_sparse_core_info()
_NW = _SC_INFO.num_cores * _SC_INFO.num_subcores
_L = _SC_INFO.num_lanes


def _router_sc_body(tok_per_w, E, s_hbm, w_hbm, s_v, w_v):
    wid = jax.lax.axis_index("s") * _SC_INFO.num_cores + jax.lax.axis_index("c")
    base = wid * tok_per_w
    pltpu.sync_copy(s_hbm.at[:, pl.ds(base, tok_per_w)], s_v)
    neg = jnp.full((_L,), -3.0e38, jnp.float32)
    for c in range(tok_per_w // _L):
        sl = pl.ds(c * _L, _L)
        svecs = [s_v[e, sl] for e in range(E)]
        m1 = svecs[0]
        for e in range(1, E):
            m1 = jnp.maximum(m1, svecs[e])
        imax = jnp.full((_L,), E, jnp.int32)
        for e in range(E):
            imax = jnp.minimum(imax,
                               jnp.where(svecs[e] == m1, e, E))
        m2 = neg
        for e in range(E):
            m2 = jnp.maximum(m2, jnp.where(imax == e, neg, svecs[e]))
        mx = jnp.maximum(m1, 0.0)
        exs = []
        for e in range(E):
            mask_e = jnp.where(svecs[e] >= m2, svecs[e], 0.0)
            exs.append(jnp.exp(mask_e - mx))
        z = exs[0]
        for e in range(1, E):
            z = z + exs[e]
        for e in range(E):
            w_v[e, sl] = exs[e] / z
    pltpu.sync_copy(w_v, w_hbm.at[:, pl.ds(base, tok_per_w)])


def _router_sc_t(scores_t):
    E, BT = scores_t.shape
    tok_per_w = BT // _NW
    mesh = plsc.VectorSubcoreMesh(core_axis_name="c", subcore_axis_name="s")
    body = functools.partial(_router_sc_body, tok_per_w, E)
    return pl.kernel(
        body,
        mesh=mesh,
        out_type=jax.ShapeDtypeStruct((E, BT), jnp.float32),
        scratch_types=[
            pltpu.VMEM((E, tok_per_w), jnp.float32),
            pltpu.VMEM((E, tok_per_w), jnp.float32),
        ],
    )(scores_t)


def _moe_grid_kernel(x_ref, wt_ref, w1_ref, b1_ref, w2_ref, b2_ref,
                     out_ref, w_scratch, we_scratch):
    e = pl.program_id(0)
    hb = pl.program_id(1)

    @pl.when((e == 0) & (hb == 0))
    def _init():
        w = jnp.swapaxes(wt_ref[...], 0, 1)
        w_scratch[...] = w
        out_ref[...] = jnp.dot(w, b2_ref[...],
                               preferred_element_type=jnp.float32)

    @pl.when(hb == 0)
    def _select_w():
        w = w_scratch[...]
        onehot = (jax.lax.broadcasted_iota(jnp.int32, w.shape, 1) == e)
        we_scratch[...] = jnp.sum(jnp.where(onehot, w, 0.0), axis=-1,
                                  keepdims=True)

    h = jnp.dot(x_ref[...], w1_ref[0], preferred_element_type=jnp.float32)
    h = jnp.maximum(h + b1_ref[0, 0], 0.0).astype(jnp.bfloat16)
    o = jnp.dot(h, w2_ref[0].astype(jnp.bfloat16),
                preferred_element_type=jnp.float32)
    out_ref[...] += we_scratch[...] * o


@functools.partial(jax.jit, static_argnames=("block_h",))
def _moe(x, W1, b1, W2, b2, Wr, br, block_h=1536):
    BT, D = x.shape
    E, _, H = W1.shape
    w_t = _router_sc_t(_logits_t(x, Wr, br))
    grid = (E, H // block_h)
    out = pl.pallas_call(
        _moe_grid_kernel,
        grid=grid,
        in_specs=[
            pl.BlockSpec((BT, D), lambda e, hb: (0, 0)),
            pl.BlockSpec((E, BT), lambda e, hb: (0, 0)),
            pl.BlockSpec((1, D, block_h), lambda e, hb: (e, 0, hb)),
            pl.BlockSpec((1, 1, block_h), lambda e, hb: (e, 0, hb)),
            pl.BlockSpec((1, block_h, D), lambda e, hb: (e, hb, 0)),
            pl.BlockSpec((E, D), lambda e, hb: (0, 0)),
        ],
        out_specs=pl.BlockSpec((BT, D), lambda e, hb: (0, 0)),
        out_shape=jax.ShapeDtypeStruct((BT, D), jnp.float32),
        scratch_shapes=[pltpu.VMEM((BT, E), jnp.float32),
                        pltpu.VMEM((BT, 1), jnp.float32)],
        compiler_params=pltpu.CompilerParams(
            dimension_semantics=("arbitrary", "arbitrary"),
            vmem_limit_bytes=67108864,
        ),
    )(x.astype(jnp.bfloat16), w_t, W1, b1.reshape(E, 1, H), W2, b2)
    return out


def kernel(inputs, W1, b1, W2, b2, Wr, br):
    B, T, D = inputs.shape
    x = inputs.reshape(B * T, D)
    out = _moe(x, W1, b1, W2, b2, Wr, br)
    return out.reshape(B, T, D)


# speedup vs baseline: 1.0068x; 1.0068x over previous
import functools

import jax
import jax.numpy as jnp
from jax.experimental import pallas as pl
from jax.experimental.pallas import tpu as pltpu
from jax.experimental.pallas import tpu_sc as plsc


def _logits_kernel(x_ref, wr_ref, br_ref, s_ref):
    s = jax.lax.dot_general(wr_ref[...], x_ref[...], (((0,), (1,)), ((), ())),
                            preferred_element_type=jnp.float32)
    s_ref[...] = s + br_ref[...]


def _logits_t(x, Wr, br):
    BT, D = x.shape
    E = Wr.shape[1]
    return pl.pallas_call(
        _logits_kernel,
        out_shape=jax.ShapeDtypeStruct((E, BT), jnp.float32),
    )(x, Wr, br.reshape(E, 1))


_SC_INFO = plsc.get_sparse_core_info()
_NW = _SC_INFO.num_cores * _SC_INFO.num_subcores
_L = _SC_INFO.num_lanes


def _router_sc_body(tok_per_w, E, s_hbm, w_hbm, s_v, w_v):
    wid = jax.lax.axis_index("s") * _SC_INFO.num_cores + jax.lax.axis_index("c")
    base = wid * tok_per_w
    pltpu.sync_copy(s_hbm.at[:, pl.ds(base, tok_per_w)], s_v)
    neg = jnp.full((_L,), -3.0e38, jnp.float32)
    for c in range(tok_per_w // _L):
        sl = pl.ds(c * _L, _L)
        svecs = [s_v[e, sl] for e in range(E)]
        m1 = svecs[0]
        for e in range(1, E):
            m1 = jnp.maximum(m1, svecs[e])
        imax = jnp.full((_L,), E, jnp.int32)
        for e in range(E):
            imax = jnp.minimum(imax,
                               jnp.where(svecs[e] == m1, e, E))
        m2 = neg
        for e in range(E):
            m2 = jnp.maximum(m2, jnp.where(imax == e, neg, svecs[e]))
        mx = jnp.maximum(m1, 0.0)
        exs = []
        for e in range(E):
            mask_e = jnp.where(svecs[e] >= m2, svecs[e], 0.0)
            exs.append(jnp.exp(mask_e - mx))
        z = exs[0]
        for e in range(1, E):
            z = z + exs[e]
        for e in range(E):
            w_v[e, sl] = exs[e] / z
    pltpu.sync_copy(w_v, w_hbm.at[:, pl.ds(base, tok_per_w)])


def _router_sc_t(scores_t):
    E, BT = scores_t.shape
    tok_per_w = BT // _NW
    mesh = plsc.VectorSubcoreMesh(core_axis_name="c", subcore_axis_name="s")
    body = functools.partial(_router_sc_body, tok_per_w, E)
    return pl.kernel(
        body,
        mesh=mesh,
        out_type=jax.ShapeDtypeStruct((E, BT), jnp.float32),
        scratch_types=[
            pltpu.VMEM((E, tok_per_w), jnp.float32),
            pltpu.VMEM((E, tok_per_w), jnp.float32),
        ],
    )(scores_t)


def _moe_grid_kernel(x_ref, wt_ref, w1_ref, b1_ref, w2_ref, b2_ref,
                     out_ref, w_scratch, we_scratch):
    e = pl.program_id(0)
    hb = pl.program_id(1)

    @pl.when((e == 0) & (hb == 0))
    def _init():
        w = jnp.swapaxes(wt_ref[...], 0, 1)
        w_scratch[...] = w
        out_ref[...] = jnp.dot(w, b2_ref[...],
                               preferred_element_type=jnp.float32)

    @pl.when(hb == 0)
    def _select_w():
        w = w_scratch[...]
        onehot = (jax.lax.broadcasted_iota(jnp.int32, w.shape, 1) == e)
        we_scratch[...] = jnp.sum(jnp.where(onehot, w, 0.0), axis=-1,
                                  keepdims=True)

    h = jnp.dot(x_ref[...], w1_ref[0], preferred_element_type=jnp.float32)
    h = jnp.maximum(h + b1_ref[0, 0], 0.0).astype(jnp.bfloat16)
    o = jnp.dot(h, w2_ref[0].astype(jnp.bfloat16),
                preferred_element_type=jnp.float32)
    out_ref[...] += we_scratch[...] * o


@functools.partial(jax.jit, static_argnames=("block_h",))
def _moe(x, W1, b1, W2, b2, Wr, br, block_h=1024):
    BT, D = x.shape
    E, _, H = W1.shape
    w_t = _router_sc_t(_logits_t(x, Wr, br))
    grid = (E, H // block_h)
    out = pl.pallas_call(
        _moe_grid_kernel,
        grid=grid,
        in_specs=[
            pl.BlockSpec((BT, D), lambda e, hb: (0, 0)),
            pl.BlockSpec((E, BT), lambda e, hb: (0, 0)),
            pl.BlockSpec((1, D, block_h), lambda e, hb: (e, 0, hb)),
            pl.BlockSpec((1, 1, block_h), lambda e, hb: (e, 0, hb)),
            pl.BlockSpec((1, block_h, D), lambda e, hb: (e, hb, 0)),
            pl.BlockSpec((E, D), lambda e, hb: (0, 0)),
        ],
        out_specs=pl.BlockSpec((BT, D), lambda e, hb: (0, 0)),
        out_shape=jax.ShapeDtypeStruct((BT, D), jnp.float32),
        scratch_shapes=[pltpu.VMEM((BT, E), jnp.float32),
                        pltpu.VMEM((BT, 1), jnp.float32)],
        compiler_params=pltpu.CompilerParams(
            dimension_semantics=("arbitrary", "arbitrary"),
        ),
    )(x, w_t, W1, b1.reshape(E, 1, H), W2, b2)
    return out


def kernel(inputs, W1, b1, W2, b2, Wr, br):
    B, T, D = inputs.shape
    x = inputs.reshape(B * T, D)
    out = _moe(x, W1, b1, W2, b2, Wr, br)
    return out.reshape(B, T, D)
